# baseline (speedup 1.0000x reference)
import functools
import math

import jax
import jax.numpy as jnp
from jax import lax
from jax.experimental import pallas as pl
from jax.experimental.pallas import tpu as pltpu
from jax.experimental.pallas import tpu_sc as plsc

L = 200
K = 16
D = 32
TAIL0 = L - K
T2 = 2 * K
WSUM = float(sum(math.exp(i - L) for i in range(L)))
WT = [math.exp(t - K) for t in range(K)]
INV = 1.0 / WSUM

SC_ROWS = 64
NC, NS = 2, 16
NW = NC * NS


def _gae_tc_kernel(srcT_ref, dstT_ref, tlT_ref, w1b_ref, b1b_ref, w2_ref,
                   b2b_ref, so_ref, do_ref):
    src = srcT_ref[...]
    dst = dstT_ref[...]
    tl = tlT_ref[...]
    rb = src.shape[1]

    acc_s = jnp.zeros((T2, rb), jnp.int32)
    acc_d = jnp.zeros((T2, rb), jnp.int32)
    for j in range(L):
        cj_s = jnp.broadcast_to(src[j:j + 1, :], (T2, rb))
        acc_s = acc_s + (tl == cj_s).astype(jnp.int32)
        cj_d = jnp.broadcast_to(dst[j:j + 1, :], (T2, rb))
        acc_d = acc_d + (tl == cj_d).astype(jnp.int32)
    valid = tl != 0
    cs = jnp.where(valid, acc_s, 0).astype(jnp.float32)
    cd = jnp.where(valid, acc_d, 0).astype(jnp.float32)

    w1b = w1b_ref[...]
    b1b = b1b_ref[...]
    accf_s = jnp.zeros((D, rb), jnp.float32)
    accf_d = jnp.zeros((D, rb), jnp.float32)
    for t in range(K):
        wt = WT[t]
        c1 = jnp.broadcast_to(cs[t:t + 1, :], (D, rb))
        c2 = jnp.broadcast_to(cd[t:t + 1, :], (D, rb))
        accf_s = accf_s + wt * (jax.nn.relu(c1 * w1b + b1b)
                                + jax.nn.relu(c2 * w1b + b1b))
        c1d = jnp.broadcast_to(cs[K + t:K + t + 1, :], (D, rb))
        c2d = jnp.broadcast_to(cd[K + t:K + t + 1, :], (D, rb))
        accf_d = accf_d + wt * (jax.nn.relu(c1d * w1b + b1b)
                                + jax.nn.relu(c2d * w1b + b1b))

    w2 = w2_ref[...]
    b2b = b2b_ref[...]
    so_ref[...] = (jnp.dot(w2, accf_s, preferred_element_type=jnp.float32)
                   * INV + b2b)
    do_ref[...] = (jnp.dot(w2, accf_d, preferred_element_type=jnp.float32)
                   * INV + b2b)


def _tc_part(src, dst, W1, b1, W2, b2):
    B = src.shape[0]
    f32 = jnp.float32
    srcT = src.T
    dstT = dst.T
    tlT = jnp.concatenate([src[:, TAIL0:], dst[:, TAIL0:]], axis=1).T

    ones = jnp.ones((1, B), f32)
    w1b = W1.reshape(D, 1) * ones
    b1b = b1.reshape(D, 1) * ones
    b2b = (2.0 * b2).reshape(D, 1) * ones

    Rb = 128
    grid = (pl.cdiv(B, Rb),)
    so, do = pl.pallas_call(
        _gae_tc_kernel,
        grid=grid,
        in_specs=[
            pl.BlockSpec((L, Rb), lambda i: (0, i)),
            pl.BlockSpec((L, Rb), lambda i: (0, i)),
            pl.BlockSpec((T2, Rb), lambda i: (0, i)),
            pl.BlockSpec((D, Rb), lambda i: (0, i)),
            pl.BlockSpec((D, Rb), lambda i: (0, i)),
            pl.BlockSpec((D, D), lambda i: (0, 0)),
            pl.BlockSpec((D, Rb), lambda i: (0, i)),
        ],
        out_specs=[pl.BlockSpec((D, Rb), lambda i: (0, i)),
                   pl.BlockSpec((D, Rb), lambda i: (0, i))],
        out_shape=[jax.ShapeDtypeStruct((D, B), f32),
                   jax.ShapeDtypeStruct((D, B), f32)],
        compiler_params=pltpu.CompilerParams(
            dimension_semantics=("parallel",)),
    )(srcT, dstT, tlT, w1b, b1b, W2, b2b)
    return so.T, do.T


LPAD = 208
NCHUNK = LPAD // 16


def _relu(x):
    return jnp.maximum(x, 0.0)


_DNUMS = lax.GatherDimensionNumbers(
    offset_dims=(), collapsed_slice_dims=(0,), start_index_map=(0,))


def _vtake(x, lane):
    idx = jnp.full((16, 1), lane, jnp.int32)
    return lax.gather(x, idx, _DNUMS, (1,),
                      mode=lax.GatherScatterMode.PROMISE_IN_BOUNDS)


def _sc_row(src_v, dst_v, w_vregs):
    w1a, w1b_, b1a, b1b_ = w_vregs
    st = src_v[pl.ds(TAIL0, 16)]
    dt = dst_v[pl.ds(TAIL0, 16)]

    css = jnp.zeros((16,), jnp.int32)
    csd = jnp.zeros((16,), jnp.int32)
    cds = jnp.zeros((16,), jnp.int32)
    cdd = jnp.zeros((16,), jnp.int32)
    for c in range(NCHUNK):
        sv = src_v[pl.ds(16 * c, 16)]
        dv = dst_v[pl.ds(16 * c, 16)]
        for lane in range(16):
            sj = _vtake(sv, lane)
            dj = _vtake(dv, lane)
            one = jnp.ones((16,), jnp.int32)
            zero = jnp.zeros((16,), jnp.int32)
            css = css + jnp.where(st == sj, one, zero)
            csd = csd + jnp.where(st == dj, one, zero)
            cds = cds + jnp.where(dt == sj, one, zero)
            cdd = cdd + jnp.where(dt == dj, one, zero)
    vs = st != 0
    vd = dt != 0
    cssf = jnp.where(vs, css, 0).astype(jnp.float32)
    csdf = jnp.where(vs, csd, 0).astype(jnp.float32)
    cdsf = jnp.where(vd, cds, 0).astype(jnp.float32)
    cddf = jnp.where(vd, cdd, 0).astype(jnp.float32)

    a0 = jnp.zeros((16,), jnp.float32)
    a1 = jnp.zeros((16,), jnp.float32)
    g0 = jnp.zeros((16,), jnp.float32)
    g1 = jnp.zeros((16,), jnp.float32)
    for t in range(K):
        wt = WT[t]
        c1 = _vtake(cssf, t)
        c2 = _vtake(csdf, t)
        a0 = a0 + wt * (_relu(c1 * w1a + b1a)
                        + _relu(c2 * w1a + b1a))
        a1 = a1 + wt * (_relu(c1 * w1b_ + b1b_)
                        + _relu(c2 * w1b_ + b1b_))
        c3 = _vtake(cdsf, t)
        c4 = _vtake(cddf, t)
        g0 = g0 + wt * (_relu(c3 * w1a + b1a)
                        + _relu(c4 * w1a + b1a))
        g1 = g1 + wt * (_relu(c3 * w1b_ + b1b_)
                        + _relu(c4 * w1b_ + b1b_))
    return a0, a1, g0, g1


def _sc_part(src, dst, W1, b1, W2, b2):
    S = src.shape[0]
    rows_per_w = S // NW
    f32 = jnp.float32
    srcp = jnp.pad(src, ((0, 0), (0, LPAD - L)))
    dstp = jnp.pad(dst, ((0, 0), (0, LPAD - L)))
    w1 = W1.reshape(D)
    w2t = W2.T.reshape(D * D)
    b2x = (2.0 * b2).reshape(D)
    mesh = plsc.VectorSubcoreMesh(core_axis_name="c", subcore_axis_name="s")

    @functools.partial(
        pl.kernel, mesh=mesh,
        out_type=[jax.ShapeDtypeStruct((S, D), f32),
                  jax.ShapeDtypeStruct((S, D), f32)],
        scratch_types=[
            pltpu.VMEM((rows_per_w, LPAD), jnp.int32),
            pltpu.VMEM((rows_per_w, LPAD), jnp.int32),
            pltpu.VMEM((rows_per_w, D), f32),
            pltpu.VMEM((rows_per_w, D), f32),
            pltpu.VMEM((D,), f32),
            pltpu.VMEM((D,), f32),
            pltpu.VMEM((D,), f32),
            pltpu.VMEM((D * D,), f32),
        ],
    )
    def sc_kernel(src_hbm, dst_hbm, w1_hbm, b1_hbm, w2t_hbm, b2_hbm,
                  so_hbm, do_hbm, src_v, dst_v, so_v, do_v, w1_v, b1_v,
                  b2_v, w2t_v):
        wid = lax.axis_index("s") * NC + lax.axis_index("c")
        base = wid * rows_per_w
        pltpu.sync_copy(src_hbm.at[pl.ds(base, rows_per_w)], src_v)
        pltpu.sync_copy(dst_hbm.at[pl.ds(base, rows_per_w)], dst_v)
        pltpu.sync_copy(w1_hbm, w1_v)
        pltpu.sync_copy(b1_hbm, b1_v)
        pltpu.sync_copy(w2t_hbm, w2t_v)
        pltpu.sync_copy(b2_hbm, b2_v)
        w1a = w1_v[pl.ds(0, 16)]
        w1b_ = w1_v[pl.ds(16, 16)]
        b1a = b1_v[pl.ds(0, 16)]
        b1b_ = b1_v[pl.ds(16, 16)]
        b2a = b2_v[pl.ds(0, 16)]
        b2b_ = b2_v[pl.ds(16, 16)]
        w_vregs = (w1a, w1b_, b1a, b1b_)

        def row_body(r, carry):
            a0, a1, g0, g1 = _sc_row(src_v.at[r], dst_v.at[r], w_vregs)
            o0 = jnp.zeros((16,), jnp.float32)
            o1 = jnp.zeros((16,), jnp.float32)
            p0 = jnp.zeros((16,), jnp.float32)
            p1 = jnp.zeros((16,), jnp.float32)
            for d in range(D):
                ad = _vtake(a0 if d < 16 else a1, d % 16)
                gd = _vtake(g0 if d < 16 else g1, d % 16)
                r0 = w2t_v[pl.ds(d * D, 16)]
                r1 = w2t_v[pl.ds(d * D + 16, 16)]
                o0 = o0 + ad * r0
                o1 = o1 + ad * r1
                p0 = p0 + gd * r0
                p1 = p1 + gd * r1
            so_v[r, pl.ds(0, 16)] = o0 * INV + b2a
            so_v[r, pl.ds(16, 16)] = o1 * INV + b2b_
            do_v[r, pl.ds(0, 16)] = p0 * INV + b2a
            do_v[r, pl.ds(16, 16)] = p1 * INV + b2b_
            return carry

        lax.fori_loop(0, rows_per_w, row_body, 0)
        pltpu.sync_copy(so_v, so_hbm.at[pl.ds(base, rows_per_w)])
        pltpu.sync_copy(do_v, do_hbm.at[pl.ds(base, rows_per_w)])

    return sc_kernel(srcp, dstp, w1, b1, w2t, b2x)


def kernel(src_padded_nodes_neighbor_ids, dst_padded_nodes_neighbor_ids,
           W1, b1, W2, b2):
    src = src_padded_nodes_neighbor_ids
    dst = dst_padded_nodes_neighbor_ids
    so_sc, do_sc = _sc_part(src[:SC_ROWS], dst[:SC_ROWS], W1, b1, W2, b2)
    so_tc, do_tc = _tc_part(src[SC_ROWS:], dst[SC_ROWS:], W1, b1, W2, b2)
    return (jnp.concatenate([so_sc, so_tc], axis=0),
            jnp.concatenate([do_sc, do_tc], axis=0))


# speedup vs baseline: 2.8331x; 2.8331x over previous
import math

import jax
import jax.numpy as jnp
from jax.experimental import pallas as pl
from jax.experimental.pallas import tpu as pltpu

L = 200
K = 16
D = 32
TAIL0 = L - K
T2 = 2 * K
WSUM = float(sum(math.exp(i - L) for i in range(L)))
WT = [math.exp(t - K) for t in range(K)]


def _gae_kernel(srcT_ref, dstT_ref, tlT_ref, w1b_ref, b1b_ref, w2_ref,
                b2b_ref, so_ref, do_ref):
    src = srcT_ref[...]
    dst = dstT_ref[...]
    tl = tlT_ref[...]
    rb = src.shape[1]

    acc_s = jnp.zeros((T2, rb), jnp.int32)
    acc_d = jnp.zeros((T2, rb), jnp.int32)
    for j in range(L):
        cj_s = jnp.broadcast_to(src[j:j + 1, :], (T2, rb))
        acc_s = acc_s + (tl == cj_s).astype(jnp.int32)
        cj_d = jnp.broadcast_to(dst[j:j + 1, :], (T2, rb))
        acc_d = acc_d + (tl == cj_d).astype(jnp.int32)
    valid = tl != 0
    cs = jnp.where(valid, acc_s, 0).astype(jnp.float32)
    cd = jnp.where(valid, acc_d, 0).astype(jnp.float32)

    w1b = w1b_ref[...]
    b1b = b1b_ref[...]
    accf_s = jnp.zeros((D, rb), jnp.float32)
    accf_d = jnp.zeros((D, rb), jnp.float32)
    for t in range(K):
        wt = WT[t]
        c1 = jnp.broadcast_to(cs[t:t + 1, :], (D, rb))
        c2 = jnp.broadcast_to(cd[t:t + 1, :], (D, rb))
        accf_s = accf_s + wt * (jax.nn.relu(c1 * w1b + b1b)
                                + jax.nn.relu(c2 * w1b + b1b))
        c1d = jnp.broadcast_to(cs[K + t:K + t + 1, :], (D, rb))
        c2d = jnp.broadcast_to(cd[K + t:K + t + 1, :], (D, rb))
        accf_d = accf_d + wt * (jax.nn.relu(c1d * w1b + b1b)
                                + jax.nn.relu(c2d * w1b + b1b))

    w2 = w2_ref[...]
    b2b = b2b_ref[...]
    inv = 1.0 / WSUM
    so_ref[...] = (jnp.dot(w2, accf_s, preferred_element_type=jnp.float32)
                   * inv + b2b)
    do_ref[...] = (jnp.dot(w2, accf_d, preferred_element_type=jnp.float32)
                   * inv + b2b)


def kernel(src_padded_nodes_neighbor_ids, dst_padded_nodes_neighbor_ids,
           W1, b1, W2, b2):
    src = src_padded_nodes_neighbor_ids
    dst = dst_padded_nodes_neighbor_ids
    B = src.shape[0]
    f32 = jnp.float32

    srcT = src.T
    dstT = dst.T
    tlT = jnp.concatenate([src[:, TAIL0:], dst[:, TAIL0:]], axis=1).T

    ones = jnp.ones((1, B), f32)
    w1b = W1.reshape(D, 1) * ones
    b1b = b1.reshape(D, 1) * ones
    b2b = (2.0 * b2).reshape(D, 1) * ones

    Rb = 256
    grid = (B // Rb,)
    so, do = pl.pallas_call(
        _gae_kernel,
        grid=grid,
        in_specs=[
            pl.BlockSpec((L, Rb), lambda i: (0, i)),
            pl.BlockSpec((L, Rb), lambda i: (0, i)),
            pl.BlockSpec((T2, Rb), lambda i: (0, i)),
            pl.BlockSpec((D, Rb), lambda i: (0, i)),
            pl.BlockSpec((D, Rb), lambda i: (0, i)),
            pl.BlockSpec((D, D), lambda i: (0, 0)),
            pl.BlockSpec((D, Rb), lambda i: (0, i)),
        ],
        out_specs=[pl.BlockSpec((D, Rb), lambda i: (0, i)),
                   pl.BlockSpec((D, Rb), lambda i: (0, i))],
        out_shape=[jax.ShapeDtypeStruct((D, B), f32),
                   jax.ShapeDtypeStruct((D, B), f32)],
        compiler_params=pltpu.CompilerParams(
            dimension_semantics=("parallel",)),
    )(srcT, dstT, tlT, w1b, b1b, W2, b2b)
    return (so.T, do.T)
